# baseline (speedup 1.0000x reference)
import functools

import jax
import jax.numpy as jnp
from jax import lax
from jax.experimental import pallas as pl
from jax.experimental.pallas import tpu as pltpu
from jax.experimental.pallas import tpu_sc as plsc

N = 10000
NP = 10240
E = 320000
F = 128
H = 64
NC = 2
NS = 16
NW = NC * NS
C = 128
NCH = 160
E_T = NCH * C
EP = NS * E_T
HH = H // 2
K = 8
NG = NCH // K
R_T = NP // NS
R_C = 128
NRC = R_T // R_C

_f32 = jnp.float32
_mesh = plsc.VectorSubcoreMesh(
    core_axis_name="c", subcore_axis_name="s", num_cores=NC, num_subcores=NS
)
_sc_params = pltpu.CompilerParams(use_tc_tiling_on_sc=False)


NCH_D = NCH // 2


@functools.partial(
    pl.kernel,
    out_type=jax.ShapeDtypeStruct((NC, NP, 16), _f32),
    mesh=_mesh,
    compiler_params=_sc_params,
    scratch_types=[
        pltpu.VMEM_SHARED((NP, 16), _f32),
        pltpu.VMEM((R_C, 16), _f32),
        pltpu.VMEM((C, 16), _f32),
        pltpu.VMEM((NCH_D, C), jnp.int32),
        pltpu.SemaphoreType.DMA,
    ],
)
def _sc_degree(dst_hbm, out_hbm, acc_sp, zbuf, ones_v, idx_d, sem):
    c = lax.axis_index("c")
    s = lax.axis_index("s")

    def _zero_row(i, _):
        zbuf[i, :] = jnp.zeros((16,), _f32)
        ones_v[i, :] = jnp.ones((16,), _f32)
        return 0

    lax.fori_loop(0, R_C, _zero_row, 0)

    pltpu.sync_copy(dst_hbm.at[s, pl.ds(c * NCH_D, NCH_D), :], idx_d)
    for r in range(NRC):
        pltpu.sync_copy(zbuf, acc_sp.at[pl.ds(s * R_T + r * R_C, R_C), :])
    plsc.subcore_barrier()

    def _group(g, _):
        descs = [
            pltpu.async_copy(ones_v, acc_sp.at[idx_d.at[g * 2 * K + b]], sem,
                             add=True)
            for b in range(2 * K)
        ]
        for d in descs:
            d.wait()
        return 0

    lax.fori_loop(0, NCH_D // (2 * K), _group, 0)
    plsc.subcore_barrier()

    for r in range(NRC):
        off = s * R_T + r * R_C
        pltpu.sync_copy(acc_sp.at[pl.ds(off, R_C), :], zbuf)
        pltpu.sync_copy(zbuf, out_hbm.at[c, pl.ds(off, R_C), :])


@functools.partial(
    pl.kernel,
    out_type=jax.ShapeDtypeStruct((NP, H), _f32),
    mesh=_mesh,
    compiler_params=_sc_params,
    scratch_types=[
        pltpu.VMEM_SHARED((NP, HH), _f32),
        pltpu.VMEM_SHARED((NP, HH), _f32),
        pltpu.VMEM((R_C, HH), _f32),
        pltpu.VMEM((K, C, HH), _f32),
        pltpu.VMEM((NCH, C), jnp.int32),
        pltpu.VMEM((NCH, C), jnp.int32),
        pltpu.SemaphoreType.DMA,
        pltpu.SemaphoreType.DMA,
    ],
)
def _sc_edge_pass(hs_hbm, src_hbm, dst_hbm, out_hbm,
                  hs_sp, acc_sp, zbuf, rows, idx_s, idx_d, gsem, ssem):
    c = lax.axis_index("c")
    s = lax.axis_index("s")
    col = c * HH

    def _zero_row(i, _):
        for k in range(HH // 16):
            zbuf[i, pl.ds(k * 16, 16)] = jnp.zeros((16,), _f32)
        return 0

    lax.fori_loop(0, R_C, _zero_row, 0)

    pltpu.sync_copy(src_hbm.at[s], idx_s)
    pltpu.sync_copy(dst_hbm.at[s], idx_d)

    for r in range(NRC):
        off = s * R_T + r * R_C
        pltpu.sync_copy(zbuf, acc_sp.at[pl.ds(off, R_C), :])
        pltpu.sync_copy(hs_hbm.at[pl.ds(off, R_C), pl.ds(col, HH)], rows.at[0])
        pltpu.sync_copy(rows.at[0], hs_sp.at[pl.ds(off, R_C), :])
    plsc.subcore_barrier()

    KB = K // 2

    def _group(g, _):
        gdA = [
            pltpu.async_copy(hs_sp.at[idx_s.at[g * K + b]], rows.at[b], gsem)
            for b in range(KB)
        ]
        for d in gdA:
            d.wait()
        sdA = [
            pltpu.async_copy(rows.at[b], acc_sp.at[idx_d.at[g * K + b]],
                             ssem, add=True)
            for b in range(KB)
        ]
        gdB = [
            pltpu.async_copy(hs_sp.at[idx_s.at[g * K + KB + b]],
                             rows.at[KB + b], gsem)
            for b in range(KB)
        ]
        for d in gdB:
            d.wait()
        sdB = [
            pltpu.async_copy(rows.at[KB + b],
                             acc_sp.at[idx_d.at[g * K + KB + b]],
                             ssem, add=True)
            for b in range(KB)
        ]
        for d in sdA + sdB:
            d.wait()
        return 0

    lax.fori_loop(0, NG, _group, 0)
    plsc.subcore_barrier()

    for r in range(NRC):
        off = s * R_T + r * R_C
        pltpu.sync_copy(acc_sp.at[pl.ds(off, R_C), :], zbuf)
        pltpu.sync_copy(zbuf, out_hbm.at[pl.ds(off, R_C), pl.ds(col, HH)])


_BLK = 1000


def _dinv_block(degp_ref):
    deg = degp_ref[0, :, 0:1] + degp_ref[1, :, 0:1] + 1.0
    return lax.rsqrt(deg)


def _tc1_body(x_ref, w_ref, degp_ref, o_ref):
    h = jnp.dot(x_ref[...], w_ref[...], preferred_element_type=_f32)
    o_ref[...] = h * _dinv_block(degp_ref)


def _tc2_body(tp_ref, hs_ref, degp_ref, b_ref, o_ref):
    dinv = _dinv_block(degp_ref)
    a = (tp_ref[...] + hs_ref[...]) * dinv + b_ref[...]
    o_ref[...] = jnp.maximum(a, 0.0) * dinv


def _tc3_body(tp_ref, hs_ref, degp_ref, w_ref, b_ref, o_ref):
    y = (tp_ref[...] + hs_ref[...]) * _dinv_block(degp_ref)
    o_ref[...] = (
        jnp.dot(y, w_ref[...], preferred_element_type=_f32) + b_ref[...]
    )


def _rows_spec(width):
    return pl.BlockSpec((_BLK, width), lambda i: (i, 0))


def _part_spec(width):
    return pl.BlockSpec((NC, _BLK, width), lambda i: (0, i, 0))


def _full_spec(shape):
    return pl.BlockSpec(shape, lambda i: tuple(0 for _ in shape))


def _tc1(x, W1, degp):
    return pl.pallas_call(
        _tc1_body,
        grid=(N // _BLK,),
        in_specs=[_rows_spec(F), _full_spec((F, H)), _part_spec(16)],
        out_specs=_rows_spec(H),
        out_shape=jax.ShapeDtypeStruct((NP, H), _f32),
    )(x, W1, degp)


def _tc2(t1p, hs1, degp, b1):
    return pl.pallas_call(
        _tc2_body,
        grid=(N // _BLK,),
        in_specs=[_rows_spec(H), _rows_spec(H), _part_spec(16), _full_spec((1, H))],
        out_specs=_rows_spec(H),
        out_shape=jax.ShapeDtypeStruct((NP, H), _f32),
    )(t1p, hs1, degp, b1)


def _tc3(t2p, hs2, degp, W2, b2):
    return pl.pallas_call(
        _tc3_body,
        grid=(N // _BLK,),
        in_specs=[
            _rows_spec(H),
            _rows_spec(H),
            _part_spec(16),
            _full_spec((H, F)),
            _full_spec((1, F)),
        ],
        out_specs=_rows_spec(F),
        out_shape=jax.ShapeDtypeStruct((N, F), _f32),
    )(t2p, hs2, degp, W2, b2)


def kernel(x, edge_index, W1, b1, W2, b2):
    ei = edge_index.astype(jnp.int32)
    pad = jnp.full((2, EP - E), N, jnp.int32)
    eip = jnp.concatenate([ei, pad], axis=1).reshape(2, NS, NCH, C)
    src, dst = eip[0], eip[1]

    degp = _sc_degree(dst)
    hs1 = _tc1(x, W1, degp)
    t1p = _sc_edge_pass(hs1, src, dst)
    hs2 = _tc2(t1p, hs1, degp, b1.reshape(1, H))
    t2p = _sc_edge_pass(hs2, src, dst)
    return _tc3(t2p, hs2, degp, W2, b2.reshape(1, F))


# speedup vs baseline: 1.0739x; 1.0739x over previous
import functools

import jax
import jax.numpy as jnp
from jax import lax
from jax.experimental import pallas as pl
from jax.experimental.pallas import tpu as pltpu
from jax.experimental.pallas import tpu_sc as plsc

N = 10000
NP = 10240
E = 320000
F = 128
H = 64
NC = 2
NS = 16
C = 128
NCH = 160
E_T = NCH * C
EP = NS * E_T
HH = H // 2
K = 8
NG = NCH // K
R_T = NP // NS
R_C = 128
NRC = R_T // R_C

_f32 = jnp.float32
_mesh = plsc.VectorSubcoreMesh(
    core_axis_name="c", subcore_axis_name="s", num_cores=NC, num_subcores=NS
)
_sc_params = pltpu.CompilerParams(
    use_tc_tiling_on_sc=False, needs_layout_passes=False
)


def _rsqrt16(x):
    i = plsc.bitcast(x, jnp.int32)
    i = jnp.full((16,), 0x5F3759DF, jnp.int32) - lax.shift_right_logical(i, 1)
    y = plsc.bitcast(i, _f32)
    for _ in range(3):
        y = y * (1.5 - 0.5 * x * y * y)
    return y


@functools.partial(
    pl.kernel,
    out_type=jax.ShapeDtypeStruct((NP, H), _f32),
    mesh=_mesh,
    compiler_params=_sc_params,
    scratch_types=[
        pltpu.VMEM_SHARED((NP, HH), _f32),
        pltpu.VMEM_SHARED((NP, HH), _f32),
        pltpu.VMEM((R_C, HH), _f32),
        pltpu.VMEM((K, C, HH), _f32),
        pltpu.VMEM((NCH, C), jnp.int32),
        pltpu.VMEM((NCH, C), jnp.int32),
        pltpu.VMEM((R_T, 16), _f32),
        pltpu.VMEM((HH,), _f32),
        pltpu.SemaphoreType.DMA,
        pltpu.SemaphoreType.DMA,
    ],
)
def _sc_mega(h1_hbm, b_hbm, src_hbm, dst_hbm, y_hbm,
             hs_sp, acc_sp, zbuf, rows, idx_s, idx_d,
             dinv_v, bv, gsem, ssem):
    c = lax.axis_index("c")
    s = lax.axis_index("s")
    col = c * HH
    sl0, sl1 = pl.ds(0, 16), pl.ds(16, 16)

    def _fill(i, _):
        z = jnp.zeros((16,), _f32)
        o = jnp.ones((16,), _f32)
        zbuf[i, sl0] = z
        zbuf[i, sl1] = z
        rows[0, i, sl0] = o
        rows[0, i, sl1] = o
        return 0

    lax.fori_loop(0, R_C, _fill, 0)

    pltpu.sync_copy(src_hbm.at[s], idx_s)
    pltpu.sync_copy(dst_hbm.at[s], idx_d)
    pltpu.sync_copy(b_hbm.at[c], bv)

    for r in range(NRC):
        pltpu.sync_copy(zbuf, acc_sp.at[pl.ds(s * R_T + r * R_C, R_C), :])
    plsc.subcore_barrier()

    def _deg_group(g, _):
        ds_ = [
            pltpu.async_copy(rows.at[0], acc_sp.at[idx_d.at[g * K + b]], ssem,
                             add=True)
            for b in range(K)
        ]
        for d in ds_:
            d.wait()
        return 0

    lax.fori_loop(0, NG, _deg_group, 0)
    plsc.subcore_barrier()

    for r in range(NRC):
        off = s * R_T + r * R_C
        pltpu.sync_copy(acc_sp.at[pl.ds(off, R_C), :], rows.at[0])
        pltpu.sync_copy(h1_hbm.at[pl.ds(off, R_C), pl.ds(col, HH)], rows.at[1])
        pltpu.sync_copy(zbuf, acc_sp.at[pl.ds(off, R_C), :])

        def _dinv_row(i, _, r=r):
            dv = _rsqrt16(rows[0, i, sl0] + 1.0)
            dinv_v[r * R_C + i, :] = dv
            rows[1, i, sl0] = rows[1, i, sl0] * dv
            rows[1, i, sl1] = rows[1, i, sl1] * dv
            return 0

        lax.fori_loop(0, R_C, _dinv_row, 0)
        pltpu.sync_copy(rows.at[1], hs_sp.at[pl.ds(off, R_C), :])
    plsc.subcore_barrier()

    def _agg_group(g, _):
        gd = [
            pltpu.async_copy(hs_sp.at[idx_s.at[g * K + b]], rows.at[b], gsem)
            for b in range(K)
        ]
        sd = []
        for b in range(K):
            gd[b].wait()
            sd.append(
                pltpu.async_copy(rows.at[b], acc_sp.at[idx_d.at[g * K + b]],
                                 ssem, add=True)
            )
        for d in sd:
            d.wait()
        return 0

    lax.fori_loop(0, NG, _agg_group, 0)
    plsc.subcore_barrier()

    b0 = bv[sl0]
    b1v = bv[sl1]
    for r in range(NRC):
        off = s * R_T + r * R_C
        pltpu.sync_copy(acc_sp.at[pl.ds(off, R_C), :], rows.at[0])
        pltpu.sync_copy(hs_sp.at[pl.ds(off, R_C), :], rows.at[1])

        def _mid_row(i, _, r=r):
            dv = dinv_v[r * R_C + i, :]
            v0 = (rows[0, i, sl0] + rows[1, i, sl0]) * dv + b0
            rows[1, i, sl0] = jnp.maximum(v0, 0.0) * dv
            v1 = (rows[0, i, sl1] + rows[1, i, sl1]) * dv + b1v
            rows[1, i, sl1] = jnp.maximum(v1, 0.0) * dv
            return 0

        lax.fori_loop(0, R_C, _mid_row, 0)
        pltpu.sync_copy(rows.at[1], hs_sp.at[pl.ds(off, R_C), :])
        pltpu.sync_copy(zbuf, acc_sp.at[pl.ds(off, R_C), :])
    plsc.subcore_barrier()

    lax.fori_loop(0, NG, _agg_group, 0)
    plsc.subcore_barrier()

    for r in range(NRC):
        off = s * R_T + r * R_C
        pltpu.sync_copy(acc_sp.at[pl.ds(off, R_C), :], rows.at[0])
        pltpu.sync_copy(hs_sp.at[pl.ds(off, R_C), :], rows.at[1])

        def _fin_row(i, _, r=r):
            dv = dinv_v[r * R_C + i, :]
            rows[0, i, sl0] = (rows[0, i, sl0] + rows[1, i, sl0]) * dv
            rows[0, i, sl1] = (rows[0, i, sl1] + rows[1, i, sl1]) * dv
            return 0

        lax.fori_loop(0, R_C, _fin_row, 0)
        pltpu.sync_copy(rows.at[0], y_hbm.at[pl.ds(off, R_C), pl.ds(col, HH)])


_BLK = 1000


def _mm1_body(x_ref, w_ref, o_ref):
    o_ref[...] = jnp.dot(x_ref[...], w_ref[...], preferred_element_type=_f32)


def _mm2_body(y_ref, w_ref, b_ref, o_ref):
    o_ref[...] = (
        jnp.dot(y_ref[...], w_ref[...], preferred_element_type=_f32)
        + b_ref[...]
    )


def _rows_spec(width):
    return pl.BlockSpec((_BLK, width), lambda i: (i, 0))


def _full_spec(shape):
    return pl.BlockSpec(shape, lambda i: tuple(0 for _ in shape))


def _tc_mm1(x, W1):
    return pl.pallas_call(
        _mm1_body,
        grid=(N // _BLK,),
        in_specs=[_rows_spec(F), _full_spec((F, H))],
        out_specs=_rows_spec(H),
        out_shape=jax.ShapeDtypeStruct((NP, H), _f32),
    )(x, W1)


def _tc_mm2(y, W2, b2):
    return pl.pallas_call(
        _mm2_body,
        grid=(N // _BLK,),
        in_specs=[_rows_spec(H), _full_spec((H, F)), _full_spec((1, F))],
        out_specs=_rows_spec(F),
        out_shape=jax.ShapeDtypeStruct((N, F), _f32),
    )(y, W2, b2)


def kernel(x, edge_index, W1, b1, W2, b2):
    ei = edge_index.astype(jnp.int32)
    pad = jnp.full((2, EP - E), N, jnp.int32)
    eip = jnp.concatenate([ei, pad], axis=1).reshape(2, NS, NCH, C)
    src, dst = eip[0], eip[1]

    h1 = _tc_mm1(x, W1)
    y = _sc_mega(h1, b1.reshape(NC, HH), src, dst)
    return _tc_mm2(y, W2, b2.reshape(1, F))


# speedup vs baseline: 1.1362x; 1.0581x over previous
import functools

import jax
import jax.numpy as jnp
from jax import lax
from jax.experimental import pallas as pl
from jax.experimental.pallas import tpu as pltpu
from jax.experimental.pallas import tpu_sc as plsc

N = 10000
NP = 10240
E = 320000
F = 128
H = 64
NC = 2
NS = 16
C = 128
NCH = 160
E_T = NCH * C
EP = NS * E_T
HH = H // 2
K = 8
NG = NCH // K
R_T = NP // NS
R_C = 128
NRC = R_T // R_C

_f32 = jnp.float32
_mesh = plsc.VectorSubcoreMesh(
    core_axis_name="c", subcore_axis_name="s", num_cores=NC, num_subcores=NS
)
_sc_params = pltpu.CompilerParams(
    use_tc_tiling_on_sc=False, needs_layout_passes=False
)


def _rsqrt16(x):
    i = plsc.bitcast(x, jnp.int32)
    i = jnp.full((16,), 0x5F3759DF, jnp.int32) - lax.shift_right_logical(i, 1)
    y = plsc.bitcast(i, _f32)
    for _ in range(3):
        y = y * (1.5 - 0.5 * x * y * y)
    return y


@functools.partial(
    pl.kernel,
    out_type=jax.ShapeDtypeStruct((NP, H), _f32),
    mesh=_mesh,
    compiler_params=_sc_params,
    scratch_types=[
        pltpu.VMEM_SHARED((NP, HH), _f32),
        pltpu.VMEM_SHARED((NP, HH), _f32),
        pltpu.VMEM((R_C, HH), _f32),
        pltpu.VMEM((K, C, HH), _f32),
        pltpu.VMEM((NCH, C), jnp.int32),
        pltpu.VMEM((NCH, C), jnp.int32),
        pltpu.VMEM((R_T, 16), _f32),
        pltpu.VMEM((HH,), _f32),
        pltpu.SemaphoreType.DMA,
        pltpu.SemaphoreType.DMA,
    ],
)
def _sc_mega(h1_hbm, b_hbm, src_hbm, dst_hbm, y_hbm,
             hs_sp, acc_sp, zbuf, rows, idx_s, idx_d,
             dinv_v, bv, gsem, ssem):
    c = lax.axis_index("c")
    s = lax.axis_index("s")
    col = c * HH
    sl0, sl1 = pl.ds(0, 16), pl.ds(16, 16)

    def _fill(i, _):
        z = jnp.zeros((16,), _f32)
        o = jnp.ones((16,), _f32)
        zbuf[i, sl0] = z
        zbuf[i, sl1] = z
        rows[0, i, sl0] = o
        rows[0, i, sl1] = o
        return 0

    lax.fori_loop(0, R_C, _fill, 0)

    pltpu.sync_copy(src_hbm.at[s], idx_s)
    pltpu.sync_copy(dst_hbm.at[s], idx_d)
    pltpu.sync_copy(b_hbm.at[c], bv)

    for r in range(NRC):
        pltpu.sync_copy(zbuf, acc_sp.at[pl.ds(s * R_T + r * R_C, R_C), :])
    plsc.subcore_barrier()

    def _deg_group(g, _):
        ds_ = [
            pltpu.async_copy(rows.at[0], acc_sp.at[idx_d.at[g * K + b]], ssem,
                             add=True)
            for b in range(K)
        ]
        for d in ds_:
            d.wait()
        return 0

    lax.fori_loop(0, NG, _deg_group, 0)
    plsc.subcore_barrier()

    for r in range(NRC):
        off = s * R_T + r * R_C
        pltpu.sync_copy(acc_sp.at[pl.ds(off, R_C), :], rows.at[0])
        pltpu.sync_copy(h1_hbm.at[pl.ds(off, R_C), pl.ds(col, HH)], rows.at[1])
        pltpu.sync_copy(zbuf, acc_sp.at[pl.ds(off, R_C), :])

        @plsc.parallel_loop(0, R_C, unroll=4)
        def _dinv_row(i, r=r):
            dv = _rsqrt16(rows[0, i, sl0] + 1.0)
            dinv_v[r * R_C + i, :] = dv
            rows[1, i, sl0] = rows[1, i, sl0] * dv
            rows[1, i, sl1] = rows[1, i, sl1] * dv
        pltpu.sync_copy(rows.at[1], hs_sp.at[pl.ds(off, R_C), :])
    plsc.subcore_barrier()

    def _agg_group(g, _):
        gd = [
            pltpu.async_copy(hs_sp.at[idx_s.at[g * K + b]], rows.at[b], gsem)
            for b in range(K)
        ]
        sd = []
        for b in range(K):
            gd[b].wait()
            sd.append(
                pltpu.async_copy(rows.at[b], acc_sp.at[idx_d.at[g * K + b]],
                                 ssem, add=True)
            )
        for d in sd:
            d.wait()
        return 0

    lax.fori_loop(0, NG, _agg_group, 0)
    plsc.subcore_barrier()

    b0 = bv[sl0]
    b1v = bv[sl1]
    for r in range(NRC):
        off = s * R_T + r * R_C
        pltpu.sync_copy(acc_sp.at[pl.ds(off, R_C), :], rows.at[0])
        pltpu.sync_copy(hs_sp.at[pl.ds(off, R_C), :], rows.at[1])

        @plsc.parallel_loop(0, R_C, unroll=4)
        def _mid_row(i, r=r):
            dv = dinv_v[r * R_C + i, :]
            v0 = (rows[0, i, sl0] + rows[1, i, sl0]) * dv + b0
            rows[1, i, sl0] = jnp.maximum(v0, 0.0) * dv
            v1 = (rows[0, i, sl1] + rows[1, i, sl1]) * dv + b1v
            rows[1, i, sl1] = jnp.maximum(v1, 0.0) * dv
        pltpu.sync_copy(rows.at[1], hs_sp.at[pl.ds(off, R_C), :])
        pltpu.sync_copy(zbuf, acc_sp.at[pl.ds(off, R_C), :])
    plsc.subcore_barrier()

    lax.fori_loop(0, NG, _agg_group, 0)
    plsc.subcore_barrier()

    for r in range(NRC):
        off = s * R_T + r * R_C
        pltpu.sync_copy(acc_sp.at[pl.ds(off, R_C), :], rows.at[0])
        pltpu.sync_copy(hs_sp.at[pl.ds(off, R_C), :], rows.at[1])

        @plsc.parallel_loop(0, R_C, unroll=4)
        def _fin_row(i, r=r):
            dv = dinv_v[r * R_C + i, :]
            rows[0, i, sl0] = (rows[0, i, sl0] + rows[1, i, sl0]) * dv
            rows[0, i, sl1] = (rows[0, i, sl1] + rows[1, i, sl1]) * dv
        pltpu.sync_copy(rows.at[0], y_hbm.at[pl.ds(off, R_C), pl.ds(col, HH)])


_BLK = 1000


def _mm1_body(x_ref, w_ref, o_ref):
    o_ref[...] = jnp.dot(x_ref[...], w_ref[...], preferred_element_type=_f32)


def _mm2_body(y_ref, w_ref, b_ref, o_ref):
    o_ref[...] = (
        jnp.dot(y_ref[...], w_ref[...], preferred_element_type=_f32)
        + b_ref[...]
    )


def _rows_spec(width):
    return pl.BlockSpec((_BLK, width), lambda i: (i, 0))


def _full_spec(shape):
    return pl.BlockSpec(shape, lambda i: tuple(0 for _ in shape))


def _tc_mm1(x, W1):
    return pl.pallas_call(
        _mm1_body,
        grid=(N // _BLK,),
        in_specs=[_rows_spec(F), _full_spec((F, H))],
        out_specs=_rows_spec(H),
        out_shape=jax.ShapeDtypeStruct((NP, H), _f32),
    )(x, W1)


def _tc_mm2(y, W2, b2):
    return pl.pallas_call(
        _mm2_body,
        grid=(N // _BLK,),
        in_specs=[_rows_spec(H), _full_spec((H, F)), _full_spec((1, F))],
        out_specs=_rows_spec(F),
        out_shape=jax.ShapeDtypeStruct((N, F), _f32),
    )(y, W2, b2)


def kernel(x, edge_index, W1, b1, W2, b2):
    ei = edge_index.astype(jnp.int32)
    pad = jnp.full((2, EP - E), N, jnp.int32)
    eip = jnp.concatenate([ei, pad], axis=1).reshape(2, NS, NCH, C)
    src, dst = eip[0], eip[1]

    h1 = _tc_mm1(x, W1)
    y = _sc_mega(h1, b1.reshape(NC, HH), src, dst)
    return _tc_mm2(y, W2, b2.reshape(1, F))


# speedup vs baseline: 1.2995x; 1.1437x over previous
import functools

import jax
import jax.numpy as jnp
from jax import lax
from jax.experimental import pallas as pl
from jax.experimental.pallas import tpu as pltpu
from jax.experimental.pallas import tpu_sc as plsc

N = 10000
NP = 10240
E = 320000
F = 128
H = 64
NC = 2
NS = 16
C = 128
NCH = 160
E_T = NCH * C
EP = NS * E_T
HH = H // 2
K = 8
NG = NCH // K
R_T = NP // NS
R_C = 128
NRC = R_T // R_C

_f32 = jnp.float32
_mesh = plsc.VectorSubcoreMesh(
    core_axis_name="c", subcore_axis_name="s", num_cores=NC, num_subcores=NS
)
_sc_params = pltpu.CompilerParams(
    use_tc_tiling_on_sc=False, needs_layout_passes=False
)


def _rsqrt16(x):
    i = plsc.bitcast(x, jnp.int32)
    i = jnp.full((16,), 0x5F3759DF, jnp.int32) - lax.shift_right_logical(i, 1)
    y = plsc.bitcast(i, _f32)
    for _ in range(3):
        y = y * (1.5 - 0.5 * x * y * y)
    return y


@functools.partial(
    pl.kernel,
    out_type=jax.ShapeDtypeStruct((NP, H), _f32),
    mesh=_mesh,
    compiler_params=_sc_params,
    scratch_types=[
        pltpu.VMEM_SHARED((NP, HH), _f32),
        pltpu.VMEM_SHARED((NP, HH), _f32),
        pltpu.VMEM((R_C, HH), _f32),
        pltpu.VMEM((K, C, HH), _f32),
        pltpu.VMEM((NCH, C), jnp.int32),
        pltpu.VMEM((NCH, C), jnp.int32),
        pltpu.VMEM((R_T, 16), _f32),
        pltpu.VMEM((HH,), _f32),
        pltpu.SemaphoreType.DMA,
        pltpu.SemaphoreType.DMA,
    ],
)
def _sc_mega(h1_hbm, b_hbm, src_hbm, dst_hbm, y_hbm,
             hs_sp, acc_sp, zbuf, rows, idx_s, idx_d,
             dinv_v, bv, gsem, ssem):
    c = lax.axis_index("c")
    s = lax.axis_index("s")
    col = c * HH
    sl0, sl1 = pl.ds(0, 16), pl.ds(16, 16)

    def _fill(i, _):
        z = jnp.zeros((16,), _f32)
        o = jnp.ones((16,), _f32)
        zbuf[i, sl0] = z
        zbuf[i, sl1] = z
        rows[0, i, sl0] = o
        rows[0, i, sl1] = o
        return 0

    lax.fori_loop(0, R_C, _fill, 0)

    pltpu.sync_copy(src_hbm.at[s], idx_s)
    pltpu.sync_copy(dst_hbm.at[s], idx_d)
    pltpu.sync_copy(b_hbm.at[c], bv)

    for r in range(NRC):
        pltpu.sync_copy(zbuf, acc_sp.at[pl.ds(s * R_T + r * R_C, R_C), :])
    plsc.subcore_barrier()

    def _drain(sem):
        pltpu.make_async_copy(
            h1_hbm.at[pl.ds(0, R_C), pl.ds(0, HH)], rows.at[K - 1], sem
        ).wait()

    for b in range(K):
        pltpu.async_copy(rows.at[0], acc_sp.at[idx_d.at[b]], ssem, add=True)

    def _deg_group(g, _):
        for b in range(K):
            _drain(ssem)
            pltpu.async_copy(rows.at[0], acc_sp.at[idx_d.at[K + g * K + b]],
                             ssem, add=True)
        return 0

    lax.fori_loop(0, NG - 1, _deg_group, 0)
    for b in range(K):
        _drain(ssem)
    plsc.subcore_barrier()

    for r in range(NRC):
        off = s * R_T + r * R_C
        pltpu.sync_copy(acc_sp.at[pl.ds(off, R_C), :], rows.at[0])
        pltpu.sync_copy(h1_hbm.at[pl.ds(off, R_C), pl.ds(col, HH)], rows.at[1])
        pltpu.sync_copy(zbuf, acc_sp.at[pl.ds(off, R_C), :])

        @plsc.parallel_loop(0, R_C, unroll=4)
        def _dinv_row(i, r=r):
            dv = _rsqrt16(rows[0, i, sl0] + 1.0)
            dinv_v[r * R_C + i, :] = dv
            rows[1, i, sl0] = rows[1, i, sl0] * dv
            rows[1, i, sl1] = rows[1, i, sl1] * dv
        pltpu.sync_copy(rows.at[1], hs_sp.at[pl.ds(off, R_C), :])
    plsc.subcore_barrier()

    L = 4

    def _run_agg():
        for _ in range(L):
            pltpu.async_copy(zbuf, acc_sp.at[idx_d.at[0]], ssem, add=True)
        for j in range(L):
            pltpu.async_copy(hs_sp.at[idx_s.at[j]], rows.at[j], gsem)

        def _step(j, b):
            _drain(gsem)
            pltpu.async_copy(rows.at[b], acc_sp.at[idx_d.at[j]], ssem,
                             add=True)
            _drain(ssem)
            pltpu.async_copy(hs_sp.at[idx_s.at[j + L]], rows.at[(b + L) % K],
                             gsem)

        def _ring(g, _):
            for b in range(K):
                _step(g * K + b, b)
            return 0

        lax.fori_loop(0, NG - 1, _ring, 0)
        for j in range((NG - 1) * K, NCH):
            b = j % K
            _drain(gsem)
            pltpu.async_copy(rows.at[b], acc_sp.at[idx_d.at[j]], ssem,
                             add=True)
            _drain(ssem)
            if j + L < NCH:
                pltpu.async_copy(hs_sp.at[idx_s.at[j + L]],
                                 rows.at[(b + L) % K], gsem)
        for _ in range(L):
            _drain(ssem)

    _run_agg()
    plsc.subcore_barrier()

    b0 = bv[sl0]
    b1v = bv[sl1]
    for r in range(NRC):
        off = s * R_T + r * R_C
        pltpu.sync_copy(acc_sp.at[pl.ds(off, R_C), :], rows.at[0])
        pltpu.sync_copy(hs_sp.at[pl.ds(off, R_C), :], rows.at[1])

        @plsc.parallel_loop(0, R_C, unroll=4)
        def _mid_row(i, r=r):
            dv = dinv_v[r * R_C + i, :]
            v0 = (rows[0, i, sl0] + rows[1, i, sl0]) * dv + b0
            rows[1, i, sl0] = jnp.maximum(v0, 0.0) * dv
            v1 = (rows[0, i, sl1] + rows[1, i, sl1]) * dv + b1v
            rows[1, i, sl1] = jnp.maximum(v1, 0.0) * dv
        pltpu.sync_copy(rows.at[1], hs_sp.at[pl.ds(off, R_C), :])
        pltpu.sync_copy(zbuf, acc_sp.at[pl.ds(off, R_C), :])
    plsc.subcore_barrier()

    _run_agg()
    plsc.subcore_barrier()

    for r in range(NRC):
        off = s * R_T + r * R_C
        pltpu.sync_copy(acc_sp.at[pl.ds(off, R_C), :], rows.at[0])
        pltpu.sync_copy(hs_sp.at[pl.ds(off, R_C), :], rows.at[1])

        @plsc.parallel_loop(0, R_C, unroll=4)
        def _fin_row(i, r=r):
            dv = dinv_v[r * R_C + i, :]
            rows[0, i, sl0] = (rows[0, i, sl0] + rows[1, i, sl0]) * dv
            rows[0, i, sl1] = (rows[0, i, sl1] + rows[1, i, sl1]) * dv
        pltpu.sync_copy(rows.at[0], y_hbm.at[pl.ds(off, R_C), pl.ds(col, HH)])


_BLK = 1000


def _mm1_body(x_ref, w_ref, o_ref):
    o_ref[...] = jnp.dot(x_ref[...], w_ref[...], preferred_element_type=_f32)


def _mm2_body(y_ref, w_ref, b_ref, o_ref):
    o_ref[...] = (
        jnp.dot(y_ref[...], w_ref[...], preferred_element_type=_f32)
        + b_ref[...]
    )


def _rows_spec(width):
    return pl.BlockSpec((_BLK, width), lambda i: (i, 0))


def _full_spec(shape):
    return pl.BlockSpec(shape, lambda i: tuple(0 for _ in shape))


def _tc_mm1(x, W1):
    return pl.pallas_call(
        _mm1_body,
        grid=(N // _BLK,),
        in_specs=[_rows_spec(F), _full_spec((F, H))],
        out_specs=_rows_spec(H),
        out_shape=jax.ShapeDtypeStruct((NP, H), _f32),
    )(x, W1)


def _tc_mm2(y, W2, b2):
    return pl.pallas_call(
        _mm2_body,
        grid=(N // _BLK,),
        in_specs=[_rows_spec(H), _full_spec((H, F)), _full_spec((1, F))],
        out_specs=_rows_spec(F),
        out_shape=jax.ShapeDtypeStruct((N, F), _f32),
    )(y, W2, b2)


def kernel(x, edge_index, W1, b1, W2, b2):
    ei = edge_index.astype(jnp.int32)
    pad = jnp.full((2, EP - E), N, jnp.int32)
    eip = jnp.concatenate([ei, pad], axis=1).reshape(2, NS, NCH, C)
    src, dst = eip[0], eip[1]

    h1 = _tc_mm1(x, W1)
    y = _sc_mega(h1, b1.reshape(NC, HH), src, dst)
    return _tc_mm2(y, W2, b2.reshape(1, F))


# speedup vs baseline: 1.3097x; 1.0078x over previous
import functools

import jax
import jax.numpy as jnp
from jax import lax
from jax.experimental import pallas as pl
from jax.experimental.pallas import tpu as pltpu
from jax.experimental.pallas import tpu_sc as plsc

N = 10000
NP = 10240
E = 320000
F = 128
H = 64
NC = 2
NS = 16
C = 128
NCH = 160
E_T = NCH * C
EP = NS * E_T
HH = H // 2
K = 8
NCH_H = NCH // 2
NGH = NCH_H // K
R_T = NP // NS
R_C = 128
NRC = R_T // R_C

_f32 = jnp.float32
_mesh = plsc.VectorSubcoreMesh(
    core_axis_name="c", subcore_axis_name="s", num_cores=NC, num_subcores=NS
)
_sc_params = pltpu.CompilerParams(
    use_tc_tiling_on_sc=False, needs_layout_passes=False
)


def _rsqrt16(x):
    i = plsc.bitcast(x, jnp.int32)
    i = jnp.full((16,), 0x5F3759DF, jnp.int32) - lax.shift_right_logical(i, 1)
    y = plsc.bitcast(i, _f32)
    for _ in range(3):
        y = y * (1.5 - 0.5 * x * y * y)
    return y


@functools.partial(
    pl.kernel,
    out_type=jax.ShapeDtypeStruct((NP, H), _f32),
    mesh=_mesh,
    compiler_params=_sc_params,
    scratch_types=[
        pltpu.VMEM_SHARED((NP, HH), _f32),
        pltpu.VMEM_SHARED((NP, HH), _f32),
        pltpu.VMEM_SHARED((NP, 16), _f32),
        pltpu.VMEM((R_C, HH), _f32),
        pltpu.VMEM((K, C, HH), _f32),
        pltpu.VMEM((NCH_H, C), jnp.int32),
        pltpu.VMEM((NCH_H, C), jnp.int32),
        pltpu.VMEM((C, 16), _f32),
        pltpu.VMEM((R_C, 16), _f32),
        pltpu.VMEM((R_T, 16), _f32),
        pltpu.VMEM((HH,), _f32),
        pltpu.SemaphoreType.DMA,
        pltpu.SemaphoreType.DMA,
    ],
)
def _sc_mega(h1_hbm, b_hbm, src_hbm, dst_hbm, y_hbm,
             hs_sp, acc_sp, deg_sp, zbuf, rows, idx_s, idx_d,
             ones16, dchunk, dinv_v, bv, gsem, ssem):
    c = lax.axis_index("c")
    s = lax.axis_index("s")
    col = c * HH
    sl0, sl1 = pl.ds(0, 16), pl.ds(16, 16)

    def _fill(i, _):
        z = jnp.zeros((16,), _f32)
        zbuf[i, sl0] = z
        zbuf[i, sl1] = z
        ones16[i, :] = jnp.ones((16,), _f32)
        dchunk[i, :] = z
        return 0

    lax.fori_loop(0, R_C, _fill, 0)

    pltpu.sync_copy(b_hbm.at[c], bv)

    def _stage_idx(half):
        pltpu.sync_copy(src_hbm.at[s, pl.ds(half * NCH_H, NCH_H), :], idx_s)
        pltpu.sync_copy(dst_hbm.at[s, pl.ds(half * NCH_H, NCH_H), :], idx_d)

    for r in range(NRC):
        pltpu.sync_copy(zbuf, acc_sp.at[pl.ds(s * R_T + r * R_C, R_C), :])
        pltpu.sync_copy(dchunk, deg_sp.at[pl.ds(s * R_T + r * R_C, R_C), :])
    plsc.subcore_barrier()

    def _drain(sem):
        pltpu.make_async_copy(
            h1_hbm.at[pl.ds(0, R_C), pl.ds(0, HH)], rows.at[K - 1], sem
        ).wait()

    def _drain16(sem):
        pltpu.make_async_copy(
            h1_hbm.at[pl.ds(0, R_C), pl.ds(0, 16)], dchunk, sem
        ).wait()

    for half in range(2):
        _stage_idx(half)
        for b in range(K):
            pltpu.async_copy(ones16, deg_sp.at[idx_d.at[b]], ssem, add=True)

        def _deg_group(g, _):
            for b in range(K):
                _drain16(ssem)
                pltpu.async_copy(ones16, deg_sp.at[idx_d.at[K + g * K + b]],
                                 ssem, add=True)
            return 0

        lax.fori_loop(0, NGH - 1, _deg_group, 0)
        for b in range(K):
            _drain16(ssem)
    plsc.subcore_barrier()

    for r in range(NRC):
        off = s * R_T + r * R_C
        pltpu.sync_copy(deg_sp.at[pl.ds(off, R_C), :], dchunk)
        pltpu.sync_copy(h1_hbm.at[pl.ds(off, R_C), pl.ds(col, HH)], rows.at[1])

        @plsc.parallel_loop(0, R_C, unroll=4)
        def _dinv_row(i, r=r):
            dv = _rsqrt16(dchunk[i, :] + 1.0)
            dinv_v[r * R_C + i, :] = dv
            rows[1, i, sl0] = rows[1, i, sl0] * dv
            rows[1, i, sl1] = rows[1, i, sl1] * dv
        pltpu.sync_copy(rows.at[1], hs_sp.at[pl.ds(off, R_C), :])
    plsc.subcore_barrier()

    L = 4

    def _run_agg_half(half):
        _stage_idx(half)
        for _ in range(L):
            pltpu.async_copy(zbuf, acc_sp.at[idx_d.at[0]], ssem, add=True)
        for j in range(L):
            pltpu.async_copy(hs_sp.at[idx_s.at[j]], rows.at[j], gsem)

        def _step(j, b):
            _drain(gsem)
            pltpu.async_copy(rows.at[b], acc_sp.at[idx_d.at[j]], ssem,
                             add=True)
            _drain(ssem)
            pltpu.async_copy(hs_sp.at[idx_s.at[j + L]], rows.at[(b + L) % K],
                             gsem)

        def _ring(g, _):
            for b in range(K):
                _step(g * K + b, b)
            return 0

        lax.fori_loop(0, NGH - 1, _ring, 0)
        for j in range((NGH - 1) * K, NCH_H):
            b = j % K
            _drain(gsem)
            pltpu.async_copy(rows.at[b], acc_sp.at[idx_d.at[j]], ssem,
                             add=True)
            _drain(ssem)
            if j + L < NCH_H:
                pltpu.async_copy(hs_sp.at[idx_s.at[j + L]],
                                 rows.at[(b + L) % K], gsem)
        for _ in range(L):
            _drain(ssem)

    def _run_agg():
        _run_agg_half(0)
        _run_agg_half(1)

    _run_agg()
    plsc.subcore_barrier()

    b0 = bv[sl0]
    b1v = bv[sl1]
    for r in range(NRC):
        off = s * R_T + r * R_C
        pltpu.sync_copy(acc_sp.at[pl.ds(off, R_C), :], rows.at[0])
        pltpu.sync_copy(hs_sp.at[pl.ds(off, R_C), :], rows.at[1])

        @plsc.parallel_loop(0, R_C, unroll=4)
        def _mid_row(i, r=r):
            dv = dinv_v[r * R_C + i, :]
            v0 = (rows[0, i, sl0] + rows[1, i, sl0]) * dv + b0
            rows[1, i, sl0] = jnp.maximum(v0, 0.0) * dv
            v1 = (rows[0, i, sl1] + rows[1, i, sl1]) * dv + b1v
            rows[1, i, sl1] = jnp.maximum(v1, 0.0) * dv
        pltpu.sync_copy(rows.at[1], hs_sp.at[pl.ds(off, R_C), :])
        pltpu.sync_copy(zbuf, acc_sp.at[pl.ds(off, R_C), :])
    plsc.subcore_barrier()

    _run_agg()
    plsc.subcore_barrier()

    for r in range(NRC):
        off = s * R_T + r * R_C
        pltpu.sync_copy(acc_sp.at[pl.ds(off, R_C), :], rows.at[0])
        pltpu.sync_copy(hs_sp.at[pl.ds(off, R_C), :], rows.at[1])

        @plsc.parallel_loop(0, R_C, unroll=4)
        def _fin_row(i, r=r):
            dv = dinv_v[r * R_C + i, :]
            rows[0, i, sl0] = (rows[0, i, sl0] + rows[1, i, sl0]) * dv
            rows[0, i, sl1] = (rows[0, i, sl1] + rows[1, i, sl1]) * dv
        pltpu.sync_copy(rows.at[0], y_hbm.at[pl.ds(off, R_C), pl.ds(col, HH)])


_BLK = 1000


def _mm1_body(x_ref, w_ref, o_ref):
    o_ref[...] = jnp.dot(x_ref[...], w_ref[...], preferred_element_type=_f32)


def _mm2_body(y_ref, w_ref, b_ref, o_ref):
    o_ref[...] = (
        jnp.dot(y_ref[...], w_ref[...], preferred_element_type=_f32)
        + b_ref[...]
    )


def _rows_spec(width):
    return pl.BlockSpec((_BLK, width), lambda i: (i, 0))


def _full_spec(shape):
    return pl.BlockSpec(shape, lambda i: tuple(0 for _ in shape))


def _tc_mm1(x, W1):
    return pl.pallas_call(
        _mm1_body,
        grid=(N // _BLK,),
        in_specs=[_rows_spec(F), _full_spec((F, H))],
        out_specs=_rows_spec(H),
        out_shape=jax.ShapeDtypeStruct((NP, H), _f32),
    )(x, W1)


def _tc_mm2(y, W2, b2):
    return pl.pallas_call(
        _mm2_body,
        grid=(N // _BLK,),
        in_specs=[_rows_spec(H), _full_spec((H, F)), _full_spec((1, F))],
        out_specs=_rows_spec(F),
        out_shape=jax.ShapeDtypeStruct((N, F), _f32),
    )(y, W2, b2)


def kernel(x, edge_index, W1, b1, W2, b2):
    ei = edge_index.astype(jnp.int32)
    pad = jnp.full((2, EP - E), N, jnp.int32)
    eip = jnp.concatenate([ei, pad], axis=1).reshape(2, NS, NCH, C)
    src, dst = eip[0], eip[1]

    h1 = _tc_mm1(x, W1)
    y = _sc_mega(h1, b1.reshape(NC, HH), src, dst)
    return _tc_mm2(y, W2, b2.reshape(1, F))
